# Initial kernel scaffold; baseline (speedup 1.0000x reference)
import jax
import jax.numpy as jnp
from jax.experimental import pallas as pl


def kernel(input_ids, segment_ids, token_table, pos_table, seg_table, ln_gamma, ln_beta):
    raise NotImplementedError("write your pallas kernel here")


# speedup vs baseline: 2.9965x; 2.9965x over previous
import functools

import jax
import jax.numpy as jnp
from jax import lax
from jax.experimental import pallas as pl
from jax.experimental.pallas import tpu as pltpu
from jax.experimental.pallas import tpu_sc as plsc

VOCAB = 100000
HIDDEN = 128
SEQ = 200
BATCH = 1024
EPS = 1e-5

NC = 2
NS = 16
NW = NC * NS
TOKENS = BATCH * SEQ
PER_W = TOKENS // NW
CHUNKS = PER_W // SEQ
NH = HIDDEN // 16


def _rsqrt(v):
    vi = lax.bitcast_convert_type(v, jnp.int32)
    yi = jnp.int32(0x5F3759DF) - lax.shift_right_logical(vi, 1)
    y = lax.bitcast_convert_type(yi, jnp.float32)
    for _ in range(3):
        y = y * (jnp.float32(1.5) - jnp.float32(0.5) * v * y * y)
    return y


def _body(ids_hbm, sids_hbm, tok_hbm, pos_hbm, seg_hbm, gam_hbm, bet_hbm,
          out_hbm, pos_v, seg_v, gam_v, bet_v, ids_v, segs_v, rows_v,
          gsem, ssem):
    wid = lax.axis_index("s") * NC + lax.axis_index("c")
    base = wid * PER_W

    pltpu.sync_copy(pos_hbm.at[pl.ds(0, SEQ)], pos_v)
    pltpu.sync_copy(seg_hbm, seg_v)
    pltpu.sync_copy(gam_hbm, gam_v)
    pltpu.sync_copy(bet_hbm, bet_v)

    gam = [gam_v[pl.ds(16 * h, 16)] for h in range(NH)]
    bet = [bet_v[pl.ds(16 * h, 16)] for h in range(NH)]

    def gather_start(buf, chunk):
        off = base + chunk * SEQ
        pltpu.sync_copy(ids_hbm.at[pl.ds(off, SEQ)], ids_v.at[buf])
        pltpu.sync_copy(sids_hbm.at[pl.ds(off, SEQ)],
                        segs_v.at[buf, pl.ds(0, SEQ)])
        pltpu.async_copy(tok_hbm.at[ids_v.at[buf]], rows_v.at[buf],
                         gsem.at[buf])

    def gather_wait(buf):
        pltpu.make_async_copy(tok_hbm.at[ids_v.at[buf]], rows_v.at[buf],
                              gsem.at[buf]).wait()

    def store_start(buf, chunk):
        off = base + chunk * SEQ
        pltpu.async_copy(rows_v.at[buf], out_hbm.at[pl.ds(off, SEQ)],
                         ssem.at[buf])

    def store_wait(buf, chunk):
        off = base + chunk * SEQ
        pltpu.make_async_copy(rows_v.at[buf], out_hbm.at[pl.ds(off, SEQ)],
                              ssem.at[buf]).wait()

    def compute(buf):
        def row(j, _):
            segi = segs_v[buf, pl.ds(j, 16)][0]
            x = []
            for h in range(NH):
                sl = pl.ds(16 * h, 16)
                t = rows_v[buf, j, sl]
                p = pos_v[j, sl]
                sg = seg_v[segi, sl]
                x.append(t + p + sg)
            s = ((x[0] + x[1]) + (x[2] + x[3])) + ((x[4] + x[5]) + (x[6] + x[7]))
            q = (((x[0] * x[0] + x[1] * x[1]) + (x[2] * x[2] + x[3] * x[3]))
                 + ((x[4] * x[4] + x[5] * x[5]) + (x[6] * x[6] + x[7] * x[7])))
            tot = jnp.sum(s)
            qt = jnp.sum(q)
            mean = tot * jnp.float32(1.0 / HIDDEN)
            var = qt * jnp.float32(1.0 / HIDDEN) - mean * mean
            rs = _rsqrt(var + jnp.float32(EPS))
            for h in range(NH):
                sl = pl.ds(16 * h, 16)
                rows_v[buf, j, sl] = (x[h] - mean) * (rs * gam[h]) + bet[h]
            return 0

        lax.fori_loop(0, SEQ, row, 0)

    gather_start(0, 0)

    def chunk_body(c, _):
        b = c % 2
        nb = 1 - b

        @pl.when(c + 1 < CHUNKS)
        def _():
            @pl.when(c >= 1)
            def _():
                store_wait(nb, c - 1)
            gather_start(nb, c + 1)

        gather_wait(b)
        compute(b)
        store_start(b, c)
        return 0

    lax.fori_loop(0, CHUNKS, chunk_body, 0)
    store_wait(0, CHUNKS - 2)
    store_wait(1, CHUNKS - 1)


@jax.jit
def _run(ids, sids, tok, pos, seg, gam, bet):
    kern = pl.kernel(
        _body,
        out_type=jax.ShapeDtypeStruct((TOKENS, HIDDEN), jnp.float32),
        mesh=plsc.VectorSubcoreMesh(core_axis_name="c", subcore_axis_name="s"),
        scratch_types=[
            pltpu.VMEM((SEQ, HIDDEN), jnp.float32),
            pltpu.VMEM((2, HIDDEN), jnp.float32),
            pltpu.VMEM((HIDDEN,), jnp.float32),
            pltpu.VMEM((HIDDEN,), jnp.float32),
            pltpu.VMEM((2, SEQ), jnp.int32),
            pltpu.VMEM((2, SEQ + 16), jnp.int32),
            pltpu.VMEM((2, SEQ, HIDDEN), jnp.float32),
            pltpu.SemaphoreType.DMA((2,)),
            pltpu.SemaphoreType.DMA((2,)),
        ],
        compiler_params=pltpu.CompilerParams(use_tc_tiling_on_sc=False,
                                             needs_layout_passes=False),
    )
    return kern(ids, sids, tok, pos, seg, gam, bet)


def kernel(input_ids, segment_ids, token_table, pos_table, seg_table,
           ln_gamma, ln_beta):
    ids = input_ids.reshape(-1).astype(jnp.int32)
    sids = segment_ids.reshape(-1).astype(jnp.int32)
    out = _run(ids, sids, token_table, pos_table, seg_table,
               ln_gamma, ln_beta)
    return out.reshape(BATCH, SEQ, HIDDEN)


# speedup vs baseline: 3.1425x; 1.0487x over previous
import functools

import jax
import jax.numpy as jnp
from jax import lax
from jax.experimental import pallas as pl
from jax.experimental.pallas import tpu as pltpu
from jax.experimental.pallas import tpu_sc as plsc

VOCAB = 100000
HIDDEN = 128
SEQ = 200
BATCH = 1024
EPS = 1e-5

NC = 2
NS = 16
NW = NC * NS
TOKENS = BATCH * SEQ
PER_W = TOKENS // NW
CHUNKS = PER_W // SEQ
NH = HIDDEN // 16


def _rsqrt(v):
    vi = lax.bitcast_convert_type(v, jnp.int32)
    yi = jnp.int32(0x5F3759DF) - lax.shift_right_logical(vi, 1)
    y = lax.bitcast_convert_type(yi, jnp.float32)
    for _ in range(2):
        y = y * (jnp.float32(1.5) - jnp.float32(0.5) * v * y * y)
    return y


def _body(ids_hbm, sids_hbm, tok_hbm, pos_hbm, seg_hbm, gam_hbm, bet_hbm,
          out_hbm, pos_v, seg_v, gam_v, bet_v, ids_v, segs_v, rows_v,
          gsem, ssem):
    wid = lax.axis_index("s") * NC + lax.axis_index("c")
    base = wid * PER_W

    pltpu.sync_copy(pos_hbm.at[pl.ds(0, SEQ)], pos_v)
    pltpu.sync_copy(seg_hbm, seg_v)
    pltpu.sync_copy(gam_hbm, gam_v)
    pltpu.sync_copy(bet_hbm, bet_v)

    gam = [gam_v[pl.ds(16 * h, 16)] for h in range(NH)]
    bet = [bet_v[pl.ds(16 * h, 16)] for h in range(NH)]

    def gather_start(buf, chunk):
        off = base + chunk * SEQ
        pltpu.sync_copy(ids_hbm.at[pl.ds(off, SEQ)], ids_v.at[buf])
        pltpu.sync_copy(sids_hbm.at[pl.ds(off, SEQ)],
                        segs_v.at[buf, pl.ds(0, SEQ)])
        pltpu.async_copy(tok_hbm.at[ids_v.at[buf]], rows_v.at[buf],
                         gsem.at[buf])

    def gather_wait(buf):
        pltpu.make_async_copy(tok_hbm.at[ids_v.at[buf]], rows_v.at[buf],
                              gsem.at[buf]).wait()

    def store_start(buf, chunk):
        off = base + chunk * SEQ
        pltpu.async_copy(rows_v.at[buf], out_hbm.at[pl.ds(off, SEQ)],
                         ssem.at[buf])

    def store_wait(buf, chunk):
        off = base + chunk * SEQ
        pltpu.make_async_copy(rows_v.at[buf], out_hbm.at[pl.ds(off, SEQ)],
                              ssem.at[buf]).wait()

    def compute(buf):
        def row(j, _):
            segi = segs_v[buf, pl.ds(j, 16)][0]
            x = []
            for h in range(NH):
                sl = pl.ds(16 * h, 16)
                t = rows_v[buf, j, sl]
                p = pos_v[j, sl]
                sg = seg_v[segi, sl]
                x.append(t + p + sg)
            s = ((x[0] + x[1]) + (x[2] + x[3])) + ((x[4] + x[5]) + (x[6] + x[7]))
            q = (((x[0] * x[0] + x[1] * x[1]) + (x[2] * x[2] + x[3] * x[3]))
                 + ((x[4] * x[4] + x[5] * x[5]) + (x[6] * x[6] + x[7] * x[7])))
            tot = jnp.sum(s)
            qt = jnp.sum(q)
            mean = tot * jnp.float32(1.0 / HIDDEN)
            var = qt * jnp.float32(1.0 / HIDDEN) - mean * mean
            rs = _rsqrt(var + jnp.float32(EPS))
            for h in range(NH):
                sl = pl.ds(16 * h, 16)
                rows_v[buf, j, sl] = (x[h] - mean) * (rs * gam[h]) + bet[h]
            return 0

        lax.fori_loop(0, SEQ, row, 0, unroll=4)

    gather_start(0, 0)

    def chunk_body(c, _):
        b = c % 2
        nb = 1 - b

        @pl.when(c + 1 < CHUNKS)
        def _():
            @pl.when(c >= 1)
            def _():
                store_wait(nb, c - 1)
            gather_start(nb, c + 1)

        gather_wait(b)
        compute(b)
        store_start(b, c)
        return 0

    lax.fori_loop(0, CHUNKS, chunk_body, 0)
    store_wait(0, CHUNKS - 2)
    store_wait(1, CHUNKS - 1)


@jax.jit
def _run(ids, sids, tok, pos, seg, gam, bet):
    kern = pl.kernel(
        _body,
        out_type=jax.ShapeDtypeStruct((TOKENS, HIDDEN), jnp.float32),
        mesh=plsc.VectorSubcoreMesh(core_axis_name="c", subcore_axis_name="s"),
        scratch_types=[
            pltpu.VMEM((SEQ, HIDDEN), jnp.float32),
            pltpu.VMEM((2, HIDDEN), jnp.float32),
            pltpu.VMEM((HIDDEN,), jnp.float32),
            pltpu.VMEM((HIDDEN,), jnp.float32),
            pltpu.VMEM((2, SEQ), jnp.int32),
            pltpu.VMEM((2, SEQ + 16), jnp.int32),
            pltpu.VMEM((2, SEQ, HIDDEN), jnp.float32),
            pltpu.SemaphoreType.DMA((2,)),
            pltpu.SemaphoreType.DMA((2,)),
        ],
        compiler_params=pltpu.CompilerParams(use_tc_tiling_on_sc=False,
                                             needs_layout_passes=False),
    )
    return kern(ids, sids, tok, pos, seg, gam, bet)


def kernel(input_ids, segment_ids, token_table, pos_table, seg_table,
           ln_gamma, ln_beta):
    ids = input_ids.reshape(-1).astype(jnp.int32)
    sids = segment_ids.reshape(-1).astype(jnp.int32)
    out = _run(ids, sids, token_table, pos_table, seg_table,
               ln_gamma, ln_beta)
    return out.reshape(BATCH, SEQ, HIDDEN)


# speedup vs baseline: 4.6235x; 1.4713x over previous
import functools

import jax
import jax.numpy as jnp
from jax import lax
from jax.experimental import pallas as pl
from jax.experimental.pallas import tpu as pltpu
from jax.experimental.pallas import tpu_sc as plsc

VOCAB = 100000
HIDDEN = 128
SEQ = 200
BATCH = 1024
EPS = 1e-5

NC = 2
NS = 16
NW = NC * NS
TOKENS = BATCH * SEQ
PER_W = TOKENS // NW
CHUNKS = PER_W // SEQ
NH = HIDDEN // 16


def _rsqrt(v):
    vi = lax.bitcast_convert_type(v, jnp.int32)
    yi = jnp.int32(0x5F3759DF) - lax.shift_right_logical(vi, 1)
    y = lax.bitcast_convert_type(yi, jnp.float32)
    for _ in range(2):
        y = y * (jnp.float32(1.5) - jnp.float32(0.5) * v * y * y)
    return y


def _body(ids_hbm, sids_hbm, tok_hbm, pos_hbm, seg_hbm, gam_hbm, bet_hbm,
          out_hbm, pos_v, seg_v, gam_v, bet_v, ids_v, segs_v, rows_v,
          gsem, ssem):
    wid = lax.axis_index("s") * NC + lax.axis_index("c")
    base = wid * PER_W

    pltpu.sync_copy(pos_hbm.at[pl.ds(0, SEQ)], pos_v)
    pltpu.sync_copy(seg_hbm, seg_v)
    pltpu.sync_copy(gam_hbm, gam_v)
    pltpu.sync_copy(bet_hbm, bet_v)

    gam = [gam_v[pl.ds(16 * h, 16)] for h in range(NH)]
    bet = [bet_v[pl.ds(16 * h, 16)] for h in range(NH)]

    def gather_start(buf, chunk):
        off = base + chunk * SEQ
        pltpu.sync_copy(ids_hbm.at[pl.ds(off, SEQ)], ids_v.at[buf])
        pltpu.sync_copy(sids_hbm.at[pl.ds(off, SEQ)],
                        segs_v.at[buf, pl.ds(0, SEQ)])
        pltpu.async_copy(tok_hbm.at[ids_v.at[buf]], rows_v.at[buf],
                         gsem.at[buf])

    def gather_wait(buf):
        pltpu.make_async_copy(tok_hbm.at[ids_v.at[buf]], rows_v.at[buf],
                              gsem.at[buf]).wait()

    def store_start(buf, chunk):
        off = base + chunk * SEQ
        pltpu.async_copy(rows_v.at[buf], out_hbm.at[pl.ds(off, SEQ)],
                         ssem.at[buf])

    def store_wait(buf, chunk):
        off = base + chunk * SEQ
        pltpu.make_async_copy(rows_v.at[buf], out_hbm.at[pl.ds(off, SEQ)],
                              ssem.at[buf]).wait()

    def compute(buf):
        @plsc.parallel_loop(0, SEQ, 1, unroll=4)
        def row(j):
            segi = segs_v[buf, pl.ds(j, 16)][0]
            x = []
            for h in range(NH):
                sl = pl.ds(16 * h, 16)
                t = rows_v[buf, j, sl]
                p = pos_v[j, sl]
                sg = seg_v[segi, sl]
                x.append(t + p + sg)
            s = ((x[0] + x[1]) + (x[2] + x[3])) + ((x[4] + x[5]) + (x[6] + x[7]))
            q = (((x[0] * x[0] + x[1] * x[1]) + (x[2] * x[2] + x[3] * x[3]))
                 + ((x[4] * x[4] + x[5] * x[5]) + (x[6] * x[6] + x[7] * x[7])))
            tot = jnp.sum(s)
            qt = jnp.sum(q)
            mean = tot * jnp.float32(1.0 / HIDDEN)
            var = qt * jnp.float32(1.0 / HIDDEN) - mean * mean
            rs = _rsqrt(var + jnp.float32(EPS))
            for h in range(NH):
                sl = pl.ds(16 * h, 16)
                rows_v[buf, j, sl] = (x[h] - mean) * (rs * gam[h]) + bet[h]

    gather_start(0, 0)

    def chunk_body(c, _):
        b = c % 2
        nb = 1 - b

        @pl.when(c + 1 < CHUNKS)
        def _():
            @pl.when(c >= 1)
            def _():
                store_wait(nb, c - 1)
            gather_start(nb, c + 1)

        gather_wait(b)
        compute(b)
        store_start(b, c)
        return 0

    lax.fori_loop(0, CHUNKS, chunk_body, 0)
    store_wait(0, CHUNKS - 2)
    store_wait(1, CHUNKS - 1)


@jax.jit
def _run(ids, sids, tok, pos, seg, gam, bet):
    kern = pl.kernel(
        _body,
        out_type=jax.ShapeDtypeStruct((TOKENS, HIDDEN), jnp.float32),
        mesh=plsc.VectorSubcoreMesh(core_axis_name="c", subcore_axis_name="s"),
        scratch_types=[
            pltpu.VMEM((SEQ, HIDDEN), jnp.float32),
            pltpu.VMEM((2, HIDDEN), jnp.float32),
            pltpu.VMEM((HIDDEN,), jnp.float32),
            pltpu.VMEM((HIDDEN,), jnp.float32),
            pltpu.VMEM((2, SEQ), jnp.int32),
            pltpu.VMEM((2, SEQ + 16), jnp.int32),
            pltpu.VMEM((2, SEQ, HIDDEN), jnp.float32),
            pltpu.SemaphoreType.DMA((2,)),
            pltpu.SemaphoreType.DMA((2,)),
        ],
        compiler_params=pltpu.CompilerParams(use_tc_tiling_on_sc=False,
                                             needs_layout_passes=False),
    )
    return kern(ids, sids, tok, pos, seg, gam, bet)


def kernel(input_ids, segment_ids, token_table, pos_table, seg_table,
           ln_gamma, ln_beta):
    ids = input_ids.reshape(-1).astype(jnp.int32)
    sids = segment_ids.reshape(-1).astype(jnp.int32)
    out = _run(ids, sids, token_table, pos_table, seg_table,
               ln_gamma, ln_beta)
    return out.reshape(BATCH, SEQ, HIDDEN)


# speedup vs baseline: 4.8330x; 1.0453x over previous
import functools

import jax
import jax.numpy as jnp
from jax import lax
from jax.experimental import pallas as pl
from jax.experimental.pallas import tpu as pltpu
from jax.experimental.pallas import tpu_sc as plsc

VOCAB = 100000
HIDDEN = 128
SEQ = 200
BATCH = 1024
EPS = 1e-5

NC = 2
NS = 16
NW = NC * NS
TOKENS = BATCH * SEQ
PER_W = TOKENS // NW
CHUNKS = PER_W // SEQ
NH = HIDDEN // 16


def _rsqrt(v):
    vi = lax.bitcast_convert_type(v, jnp.int32)
    yi = jnp.int32(0x5F3759DF) - lax.shift_right_logical(vi, 1)
    y = lax.bitcast_convert_type(yi, jnp.float32)
    for _ in range(2):
        y = y * (jnp.float32(1.5) - jnp.float32(0.5) * v * y * y)
    return y


def _body(ids_hbm, sids_hbm, tok_hbm, pos_hbm, seg_hbm, gam_hbm, bet_hbm,
          out_hbm, pos_v, seg_v, gam_v, bet_v, ids_v, segs_v, rows_v,
          gsem, ssem):
    wid = lax.axis_index("s") * NC + lax.axis_index("c")
    base = wid * PER_W

    pltpu.sync_copy(pos_hbm.at[pl.ds(0, SEQ)], pos_v)
    pltpu.sync_copy(seg_hbm, seg_v)
    pltpu.sync_copy(gam_hbm, gam_v)
    pltpu.sync_copy(bet_hbm, bet_v)

    gam = [gam_v[pl.ds(16 * h, 16)] for h in range(NH)]
    bet = [bet_v[pl.ds(16 * h, 16)] for h in range(NH)]

    def gather_start(buf, chunk):
        off = base + chunk * SEQ
        pltpu.sync_copy(ids_hbm.at[pl.ds(off, SEQ)], ids_v.at[buf])
        pltpu.sync_copy(sids_hbm.at[pl.ds(off, SEQ)],
                        segs_v.at[buf, pl.ds(0, SEQ)])
        pltpu.async_copy(tok_hbm.at[ids_v.at[buf]], rows_v.at[buf],
                         gsem.at[buf])

    def gather_wait(buf):
        pltpu.make_async_copy(tok_hbm.at[ids_v.at[buf]], rows_v.at[buf],
                              gsem.at[buf]).wait()

    def store_start(buf, chunk):
        off = base + chunk * SEQ
        pltpu.async_copy(rows_v.at[buf], out_hbm.at[pl.ds(off, SEQ)],
                         ssem.at[buf])

    def store_wait(buf, chunk):
        off = base + chunk * SEQ
        pltpu.make_async_copy(rows_v.at[buf], out_hbm.at[pl.ds(off, SEQ)],
                              ssem.at[buf]).wait()

    def compute(buf):
        @plsc.parallel_loop(0, SEQ, 1, unroll=8)
        def row(j):
            segi = segs_v[buf, pl.ds(j, 16)][0]
            x = []
            for h in range(NH):
                sl = pl.ds(16 * h, 16)
                t = rows_v[buf, j, sl]
                p = pos_v[j, sl]
                sg = seg_v[segi, sl]
                x.append(t + p + sg)
            s = ((x[0] + x[1]) + (x[2] + x[3])) + ((x[4] + x[5]) + (x[6] + x[7]))
            q = (((x[0] * x[0] + x[1] * x[1]) + (x[2] * x[2] + x[3] * x[3]))
                 + ((x[4] * x[4] + x[5] * x[5]) + (x[6] * x[6] + x[7] * x[7])))
            tot = jnp.sum(s)
            qt = jnp.sum(q)
            mean = tot * jnp.float32(1.0 / HIDDEN)
            var = qt * jnp.float32(1.0 / HIDDEN) - mean * mean
            rs = _rsqrt(var + jnp.float32(EPS))
            for h in range(NH):
                sl = pl.ds(16 * h, 16)
                rows_v[buf, j, sl] = (x[h] - mean) * (rs * gam[h]) + bet[h]

    gather_start(0, 0)

    def chunk_body(c, _):
        b = c % 2
        nb = 1 - b

        @pl.when(c + 1 < CHUNKS)
        def _():
            @pl.when(c >= 1)
            def _():
                store_wait(nb, c - 1)
            gather_start(nb, c + 1)

        gather_wait(b)
        compute(b)
        store_start(b, c)
        return 0

    lax.fori_loop(0, CHUNKS, chunk_body, 0)
    store_wait(0, CHUNKS - 2)
    store_wait(1, CHUNKS - 1)


@jax.jit
def _run(ids, sids, tok, pos, seg, gam, bet):
    kern = pl.kernel(
        _body,
        out_type=jax.ShapeDtypeStruct((TOKENS, HIDDEN), jnp.float32),
        mesh=plsc.VectorSubcoreMesh(core_axis_name="c", subcore_axis_name="s"),
        scratch_types=[
            pltpu.VMEM((SEQ, HIDDEN), jnp.float32),
            pltpu.VMEM((2, HIDDEN), jnp.float32),
            pltpu.VMEM((HIDDEN,), jnp.float32),
            pltpu.VMEM((HIDDEN,), jnp.float32),
            pltpu.VMEM((2, SEQ), jnp.int32),
            pltpu.VMEM((2, SEQ + 16), jnp.int32),
            pltpu.VMEM((2, SEQ, HIDDEN), jnp.float32),
            pltpu.SemaphoreType.DMA((2,)),
            pltpu.SemaphoreType.DMA((2,)),
        ],
        compiler_params=pltpu.CompilerParams(use_tc_tiling_on_sc=False,
                                             needs_layout_passes=False),
    )
    return kern(ids, sids, tok, pos, seg, gam, bet)


def kernel(input_ids, segment_ids, token_table, pos_table, seg_table,
           ln_gamma, ln_beta):
    ids = input_ids.reshape(-1).astype(jnp.int32)
    sids = segment_ids.reshape(-1).astype(jnp.int32)
    out = _run(ids, sids, token_table, pos_table, seg_table,
               ln_gamma, ln_beta)
    return out.reshape(BATCH, SEQ, HIDDEN)


# speedup vs baseline: 5.0481x; 1.0445x over previous
import functools

import jax
import jax.numpy as jnp
from jax import lax
from jax.experimental import pallas as pl
from jax.experimental.pallas import tpu as pltpu
from jax.experimental.pallas import tpu_sc as plsc

VOCAB = 100000
HIDDEN = 128
SEQ = 200
BATCH = 1024
EPS = 1e-5

NC = 2
NS = 16
NW = NC * NS
TOKENS = BATCH * SEQ
PER_W = TOKENS // NW
CHUNKS = PER_W // SEQ
NH = HIDDEN // 16


def _rsqrt(v):
    vi = lax.bitcast_convert_type(v, jnp.int32)
    yi = jnp.int32(0x5F3759DF) - lax.shift_right_logical(vi, 1)
    y = lax.bitcast_convert_type(yi, jnp.float32)
    for _ in range(2):
        y = y * (jnp.float32(1.5) - jnp.float32(0.5) * v * y * y)
    return y


def _body(ids_hbm, sids_hbm, tok_hbm, pos_hbm, seg_hbm, gam_hbm, bet_hbm,
          out_hbm, ps_v, gam_v, bet_v, ids_v, segs_v, rows_v, gsem, ssem):
    wid = lax.axis_index("s") * NC + lax.axis_index("c")
    base = wid * PER_W

    pltpu.sync_copy(ids_hbm.at[pl.ds(base, PER_W)], ids_v)
    pltpu.sync_copy(sids_hbm.at[pl.ds(base, PER_W)],
                    segs_v.at[pl.ds(0, PER_W)])
    pltpu.sync_copy(pos_hbm.at[pl.ds(0, SEQ)], ps_v.at[0])
    pltpu.sync_copy(gam_hbm, gam_v)
    pltpu.sync_copy(bet_hbm, bet_v)

    gam = [gam_v[pl.ds(16 * h, 16)] for h in range(NH)]
    bet = [bet_v[pl.ds(16 * h, 16)] for h in range(NH)]

    s_rows = [[None] * NH for _ in range(2)]
    pltpu.sync_copy(seg_hbm, rows_v.at[0, pl.ds(0, 2)])
    for si in range(2):
        for h in range(NH):
            s_rows[si][h] = rows_v[0, si, pl.ds(16 * h, 16)]

    @plsc.parallel_loop(0, SEQ, 1, unroll=4)
    def buildrow(j):
        for h in range(NH):
            sl = pl.ds(16 * h, 16)
            p = ps_v[0, j, sl]
            ps_v[1, j, sl] = p + s_rows[1][h]
            ps_v[0, j, sl] = p + s_rows[0][h]

    def gather_start(buf, chunk):
        pltpu.async_copy(
            tok_hbm.at[ids_v.at[pl.ds(chunk * SEQ, SEQ)]],
            rows_v.at[buf], gsem.at[buf])

    def gather_wait(buf, chunk):
        pltpu.make_async_copy(
            tok_hbm.at[ids_v.at[pl.ds(chunk * SEQ, SEQ)]],
            rows_v.at[buf], gsem.at[buf]).wait()

    def store_start(buf, chunk):
        off = base + chunk * SEQ
        pltpu.async_copy(rows_v.at[buf], out_hbm.at[pl.ds(off, SEQ)],
                         ssem.at[buf])

    def store_wait(buf, chunk):
        off = base + chunk * SEQ
        pltpu.make_async_copy(rows_v.at[buf], out_hbm.at[pl.ds(off, SEQ)],
                              ssem.at[buf]).wait()

    def compute(buf, chunk):
        @plsc.parallel_loop(0, SEQ, 1, unroll=8)
        def row(j):
            segi = segs_v[pl.ds(chunk * SEQ + j, 16)][0]
            x = []
            for h in range(NH):
                sl = pl.ds(16 * h, 16)
                x.append(rows_v[buf, j, sl] + ps_v[segi, j, sl])
            s = ((x[0] + x[1]) + (x[2] + x[3])) + ((x[4] + x[5]) + (x[6] + x[7]))
            q = (((x[0] * x[0] + x[1] * x[1]) + (x[2] * x[2] + x[3] * x[3]))
                 + ((x[4] * x[4] + x[5] * x[5]) + (x[6] * x[6] + x[7] * x[7])))
            tot = jnp.sum(s)
            qt = jnp.sum(q)
            mean = tot * jnp.float32(1.0 / HIDDEN)
            var = qt * jnp.float32(1.0 / HIDDEN) - mean * mean
            rs = _rsqrt(var + jnp.float32(EPS))
            for h in range(NH):
                sl = pl.ds(16 * h, 16)
                rows_v[buf, j, sl] = (x[h] - mean) * (rs * gam[h]) + bet[h]

    gather_start(0, 0)

    def chunk_body(c, _):
        b = c % 2
        nb = 1 - b

        @pl.when(c + 1 < CHUNKS)
        def _():
            @pl.when(c >= 1)
            def _():
                store_wait(nb, c - 1)
            gather_start(nb, c + 1)

        gather_wait(b, c)
        compute(b, c)
        store_start(b, c)
        return 0

    lax.fori_loop(0, CHUNKS, chunk_body, 0)
    store_wait(0, CHUNKS - 2)
    store_wait(1, CHUNKS - 1)


@jax.jit
def _run(ids, sids, tok, pos, seg, gam, bet):
    kern = pl.kernel(
        _body,
        out_type=jax.ShapeDtypeStruct((TOKENS, HIDDEN), jnp.float32),
        mesh=plsc.VectorSubcoreMesh(core_axis_name="c", subcore_axis_name="s"),
        scratch_types=[
            pltpu.VMEM((2, SEQ, HIDDEN), jnp.float32),
            pltpu.VMEM((HIDDEN,), jnp.float32),
            pltpu.VMEM((HIDDEN,), jnp.float32),
            pltpu.VMEM((PER_W,), jnp.int32),
            pltpu.VMEM((PER_W + 16,), jnp.int32),
            pltpu.VMEM((2, SEQ, HIDDEN), jnp.float32),
            pltpu.SemaphoreType.DMA((2,)),
            pltpu.SemaphoreType.DMA((2,)),
        ],
        compiler_params=pltpu.CompilerParams(use_tc_tiling_on_sc=False,
                                             needs_layout_passes=False),
    )
    return kern(ids, sids, tok, pos, seg, gam, bet)


def kernel(input_ids, segment_ids, token_table, pos_table, seg_table,
           ln_gamma, ln_beta):
    ids = input_ids.reshape(-1).astype(jnp.int32)
    sids = segment_ids.reshape(-1).astype(jnp.int32)
    out = _run(ids, sids, token_table, pos_table, seg_table,
               ln_gamma, ln_beta)
    return out.reshape(BATCH, SEQ, HIDDEN)
